# baseline (speedup 1.0000x reference)
import jax
import jax.numpy as jnp
from jax.experimental import pallas as pl
from jax.experimental.pallas import tpu as pltpu


_CHUNK = 2048


def _body(desc_hbm, z_ref, w0_ref, b0_ref, w1_ref, out_ref, vbuf, sems):
    i = pl.program_id(0)
    nb = pl.num_programs(0)

    @pl.when(i == 0)
    def _():
        pltpu.make_async_copy(
            desc_hbm.at[pl.ds(0, _CHUNK), :], vbuf.at[0], sems.at[0]).start()

    @pl.when(i + 1 < nb)
    def _():
        slot_next = (i + 1) % 2
        pltpu.make_async_copy(
            desc_hbm.at[pl.ds((i + 1) * _CHUNK, _CHUNK), :],
            vbuf.at[slot_next], sems.at[slot_next]).start()

    slot = i % 2
    pltpu.make_async_copy(
        desc_hbm.at[pl.ds(i * _CHUNK, _CHUNK), :],
        vbuf.at[slot], sems.at[slot]).wait()

    q_bf = vbuf[slot].astype(jnp.bfloat16)
    z_row = z_ref[...].reshape(1, -1)
    tt = jax.lax.broadcasted_iota(jnp.int32, (16, 1), 0)
    m16 = (tt % 8 == z_row).astype(jnp.float32)
    mc = m16.T
    mc_bf = mc.astype(jnp.bfloat16)

    zpad = jnp.zeros((4, 128), jnp.float32)
    b0p = jnp.concatenate([b0_ref[...], zpad], axis=0).astype(jnp.bfloat16)
    w1f = w1_ref[...]
    w1hi4 = w1f.astype(jnp.bfloat16)
    w1lo4 = (w1f - w1hi4.astype(jnp.float32)).astype(jnp.bfloat16)
    zpad_bf = zpad.astype(jnp.bfloat16)
    w1hilo = jnp.concatenate([w1hi4, zpad_bf, w1lo4, zpad_bf], axis=0)

    acc = jnp.dot(mc_bf[:, :8], b0p, preferred_element_type=jnp.float32)
    for t in range(4):
        a_t = jnp.dot(q_bf, w0_ref[t].astype(jnp.bfloat16),
                      preferred_element_type=jnp.float32)
        acc = acc + a_t * mc[:, t:t + 1]
    th = jnp.tanh(acc)
    w1sel = jnp.dot(mc_bf, w1hilo, preferred_element_type=jnp.float32)
    out_ref[...] = jnp.sum(th * w1sel, axis=0, keepdims=True)[None]


def kernel(descriptors, gradients, grad_index, positions, Z, box, W0, b0, W1, b1):
    n, d = descriptors.shape
    t, _, h = W0.shape
    chunk = min(_CHUNK, n)
    nb = n // chunk
    z3 = Z.astype(jnp.int32).reshape(nb, 1, chunk)

    out = pl.pallas_call(
        _body,
        grid=(nb,),
        in_specs=[
            pl.BlockSpec(memory_space=pl.ANY),
            pl.BlockSpec((1, 1, chunk), lambda i: (i, 0, 0)),
            pl.BlockSpec((t, d, h), lambda i: (0, 0, 0)),
            pl.BlockSpec((t, h), lambda i: (0, 0)),
            pl.BlockSpec((t, h), lambda i: (0, 0)),
        ],
        out_specs=pl.BlockSpec((1, 1, h), lambda i: (i, 0, 0)),
        out_shape=jax.ShapeDtypeStruct((nb, 1, h), jnp.float32),
        scratch_shapes=[
            pltpu.VMEM((2, _CHUNK, 128), jnp.float32),
            pltpu.SemaphoreType.DMA((2,)),
        ],
    )(descriptors, z3, W0, b0, W1)
    return -(jnp.sum(out) + n * b1)


# speedup vs baseline: 1.0627x; 1.0627x over previous
import jax
import jax.numpy as jnp
from jax.experimental import pallas as pl
from jax.experimental.pallas import tpu as pltpu


_BLOCK = 4096
_STRIP = 512


def _body(desc_ref, z_ref, w0_ref, b0_ref, w1_ref, out_ref):
    z_row = z_ref[...].reshape(1, -1)
    tt = jax.lax.broadcasted_iota(jnp.int32, (16, 1), 0)
    m16 = (tt % 8 == z_row).astype(jnp.float32)
    mc = m16.T

    zpad = jnp.zeros((4, 128), jnp.float32)
    b0p = jnp.concatenate([b0_ref[...], zpad], axis=0).astype(jnp.bfloat16)
    w1f = w1_ref[...]
    w1hi4 = w1f.astype(jnp.bfloat16)
    w1lo4 = (w1f - w1hi4.astype(jnp.float32)).astype(jnp.bfloat16)
    zpad_bf = zpad.astype(jnp.bfloat16)
    w1hilo = jnp.concatenate([w1hi4, zpad_bf, w1lo4, zpad_bf], axis=0)
    w0_bf = [w0_ref[t].astype(jnp.bfloat16) for t in range(4)]

    block = desc_ref.shape[0]
    evec = jnp.zeros((1, 128), jnp.float32)
    for s in range(block // _STRIP):
        qs_bf = desc_ref[s * _STRIP:(s + 1) * _STRIP, :].astype(jnp.bfloat16)
        mcs = mc[s * _STRIP:(s + 1) * _STRIP, :]
        mcs_bf = mcs.astype(jnp.bfloat16)
        acc = jnp.dot(mcs_bf[:, :8], b0p, preferred_element_type=jnp.float32)
        for t in range(4):
            a_t = jnp.dot(qs_bf, w0_bf[t], preferred_element_type=jnp.float32)
            acc = acc + a_t * mcs[:, t:t + 1]
        th = jnp.tanh(acc)
        w1sel = jnp.dot(mcs_bf, w1hilo, preferred_element_type=jnp.float32)
        evec = evec + jnp.sum(th * w1sel, axis=0, keepdims=True)

    out_ref[...] = evec[None]


def kernel(descriptors, gradients, grad_index, positions, Z, box, W0, b0, W1, b1):
    n, d = descriptors.shape
    t, _, h = W0.shape
    block = min(_BLOCK, n)
    nb = n // block
    z3 = Z.astype(jnp.int32).reshape(nb, 1, block)

    out = pl.pallas_call(
        _body,
        grid=(nb,),
        in_specs=[
            pl.BlockSpec((block, d), lambda i: (i, 0)),
            pl.BlockSpec((1, 1, block), lambda i: (i, 0, 0)),
            pl.BlockSpec((t, d, h), lambda i: (0, 0, 0)),
            pl.BlockSpec((t, h), lambda i: (0, 0)),
            pl.BlockSpec((t, h), lambda i: (0, 0)),
        ],
        out_specs=pl.BlockSpec((1, 1, h), lambda i: (i, 0, 0)),
        out_shape=jax.ShapeDtypeStruct((nb, 1, h), jnp.float32),
        compiler_params=pltpu.CompilerParams(
            dimension_semantics=("parallel",)),
    )(descriptors, z3, W0, b0, W1)
    return -(jnp.sum(out) + n * b1)
